# baseline (speedup 1.0000x reference)
import jax
import jax.numpy as jnp
import numpy as np
from jax import lax
from jax.experimental import pallas as pl
from jax.experimental.pallas import tpu as pltpu
from jax.experimental.pallas import tpu_sc as plsc

NUM_BANDS_K = 200
TOP_K_K = 30
R_TOTAL = 16 * 128 * 128
NC, NS = 2, 16
NW = NC * NS
ROWS_PER_W = R_TOTAL // NW
CHUNK_ROWS = 256
NCHUNK = ROWS_PER_W // CHUNK_ROWS
IN_WORDS = CHUNK_ROWS * NUM_BANDS_K
OUT_WORDS = CHUNK_ROWS * TOP_K_K
OUT_VECS = OUT_WORDS // 16
INNER_UNROLL = 15
INNER_ITERS = OUT_VECS // INNER_UNROLL


def _sc_body(x_hbm, pat_hbm, out_hbm, pat_v, in_v, out_v):
    c = lax.axis_index("c")
    s = lax.axis_index("s")
    wid = s * NC + c

    pltpu.sync_copy(pat_hbm, pat_v)
    row_base = wid * ROWS_PER_W

    def chunk_body(g, carry):
        r0 = row_base + g * CHUNK_ROWS
        pltpu.sync_copy(x_hbm.at[pl.ds(r0 * NUM_BANDS_K, IN_WORDS)], in_v)

        def grp(j, carry2):
            base = j * (INNER_UNROLL * 16)
            for v in range(INNER_UNROLL):
                off = base + v * 16
                idx = pat_v[pl.ds(off, 16)]
                out_v[pl.ds(off, 16)] = plsc.load_gather(in_v, [idx])
            return carry2

        lax.fori_loop(0, INNER_ITERS, grp, 0, unroll=False)
        pltpu.sync_copy(out_v, out_hbm.at[pl.ds(r0 * TOP_K_K, OUT_WORDS)])
        return carry

    lax.fori_loop(0, NCHUNK, chunk_body, 0, unroll=False)


@jax.jit
def kernel(x, selected_bands):
    xf = x.reshape(R_TOTAL * NUM_BANDS_K)
    sel = selected_bands.astype(jnp.int32)
    q = np.arange(OUT_WORDS)
    pat = jnp.asarray(q // TOP_K_K, jnp.int32) * NUM_BANDS_K + sel[q % TOP_K_K]

    mesh = plsc.VectorSubcoreMesh(
        core_axis_name="c", subcore_axis_name="s", num_cores=NC,
        num_subcores=NS)
    fn = pl.kernel(
        _sc_body,
        out_type=jax.ShapeDtypeStruct((R_TOTAL * TOP_K_K,), jnp.float32),
        mesh=mesh,
        compiler_params=pltpu.CompilerParams(needs_layout_passes=False),
        scratch_types=[
            pltpu.VMEM((OUT_WORDS,), jnp.int32),
            pltpu.VMEM((IN_WORDS,), jnp.float32),
            pltpu.VMEM((OUT_WORDS,), jnp.float32),
        ],
    )
    out = fn(xf, pat)
    return out.reshape(16, 128, 128, TOP_K_K)


# speedup vs baseline: 1.1743x; 1.1743x over previous
import jax
import jax.numpy as jnp
import numpy as np
from jax import lax
from jax.experimental import pallas as pl
from jax.experimental.pallas import tpu as pltpu
from jax.experimental.pallas import tpu_sc as plsc

NUM_BANDS_K = 200
TOP_K_K = 30
R_TOTAL = 16 * 128 * 128
NC, NS = 2, 16
NW = NC * NS
ROWS_PER_W = R_TOTAL // NW
CHUNK_ROWS = 256
NCHUNK = ROWS_PER_W // CHUNK_ROWS
IN_WORDS = CHUNK_ROWS * NUM_BANDS_K
OUT_WORDS = CHUNK_ROWS * TOP_K_K
OUT_VECS = OUT_WORDS // 16


def _sc_body(x_hbm, pat_hbm, out_hbm, pat_v, in_v0, in_v1, out_v0, out_v1,
             rsem, wsem):
    in_bufs = (in_v0, in_v1)
    out_bufs = (out_v0, out_v1)
    c = lax.axis_index("c")
    s = lax.axis_index("s")
    wid = s * NC + c

    pltpu.sync_copy(pat_hbm, pat_v)
    row_base = wid * ROWS_PER_W

    def rd_start(g, b):
        off = (row_base + g * CHUNK_ROWS) * NUM_BANDS_K
        pltpu.async_copy(x_hbm.at[pl.ds(off, IN_WORDS)], in_bufs[b],
                         rsem.at[b])

    def rd_wait(b):
        pltpu.make_async_copy(x_hbm.at[pl.ds(0, IN_WORDS)], in_bufs[b],
                              rsem.at[b]).wait()

    def wr_start(g, b):
        off = (row_base + g * CHUNK_ROWS) * TOP_K_K
        pltpu.async_copy(out_bufs[b], out_hbm.at[pl.ds(off, OUT_WORDS)],
                         wsem.at[b])

    def wr_wait(b):
        pltpu.make_async_copy(out_bufs[b], out_hbm.at[pl.ds(0, OUT_WORDS)],
                              wsem.at[b]).wait()

    def compute(b):
        in_ref = in_bufs[b]
        out_ref = out_bufs[b]

        @plsc.parallel_loop(0, OUT_VECS, 1, unroll=8)
        def _(i):
            off = i * 16
            idx = pat_v[pl.ds(off, 16)]
            out_ref[pl.ds(off, 16)] = plsc.load_gather(in_ref, [idx])

    rd_start(0, 0)

    def pair(t, carry):
        for b in (0, 1):
            g = 2 * t + b

            @pl.when(g + 1 < NCHUNK)
            def _():
                rd_start(g + 1, 1 - b)

            rd_wait(b)

            @pl.when(g >= 2)
            def _():
                wr_wait(b)

            compute(b)
            wr_start(g, b)
        return carry

    lax.fori_loop(0, NCHUNK // 2, pair, 0, unroll=False)
    wr_wait(0)
    wr_wait(1)


@jax.jit
def kernel(x, selected_bands):
    xf = x.reshape(R_TOTAL * NUM_BANDS_K)
    sel = selected_bands.astype(jnp.int32)
    q = np.arange(OUT_WORDS)
    pat = jnp.asarray(q // TOP_K_K, jnp.int32) * NUM_BANDS_K + sel[q % TOP_K_K]

    mesh = plsc.VectorSubcoreMesh(
        core_axis_name="c", subcore_axis_name="s", num_cores=NC,
        num_subcores=NS)
    fn = pl.kernel(
        _sc_body,
        out_type=jax.ShapeDtypeStruct((R_TOTAL * TOP_K_K,), jnp.float32),
        mesh=mesh,
        compiler_params=pltpu.CompilerParams(needs_layout_passes=False),
        scratch_types=[
            pltpu.VMEM((OUT_WORDS,), jnp.int32),
            pltpu.VMEM((IN_WORDS,), jnp.float32),
            pltpu.VMEM((IN_WORDS,), jnp.float32),
            pltpu.VMEM((OUT_WORDS,), jnp.float32),
            pltpu.VMEM((OUT_WORDS,), jnp.float32),
            pltpu.SemaphoreType.DMA((2,)),
            pltpu.SemaphoreType.DMA((2,)),
        ],
    )
    out = fn(xf, pat)
    return out.reshape(16, 128, 128, TOP_K_K)


# speedup vs baseline: 2.6736x; 2.2767x over previous
import jax
import jax.numpy as jnp
import numpy as np
from jax import lax
from jax.experimental import pallas as pl
from jax.experimental.pallas import tpu as pltpu
from jax.experimental.pallas import tpu_sc as plsc

NUM_BANDS_K = 200
TOP_K_K = 30
BLK_ROWS = 128
NBLK = 16 * 128
NC, NS = 2, 16
NW = NC * NS
BLK_PER_W = NBLK // NW
OUT_WORDS = BLK_ROWS * TOP_K_K
OUT_VECS = OUT_WORDS // 16


def _sc_body(x_hbm, patr_hbm, pcs_hbm, pco_hbm, out_hbm,
             patr_v, pcs_v, pco_v, in_v0, in_v1, out_v0, out_v1, rsem, wsem):
    in_bufs = (in_v0, in_v1)
    out_bufs = (out_v0, out_v1)
    c = lax.axis_index("c")
    s = lax.axis_index("s")
    wid = s * NC + c

    pltpu.sync_copy(patr_hbm, patr_v)
    pltpu.sync_copy(pcs_hbm, pcs_v)
    pltpu.sync_copy(pco_hbm, pco_v)
    blk_base = wid * BLK_PER_W

    def rd_start(g, b):
        pltpu.async_copy(x_hbm.at[blk_base + g], in_bufs[b], rsem.at[b])

    def rd_wait(b):
        pltpu.make_async_copy(x_hbm.at[0], in_bufs[b], rsem.at[b]).wait()

    def wr_start(g, b):
        pltpu.async_copy(out_bufs[b], out_hbm.at[blk_base + g], wsem.at[b])

    def wr_wait(b):
        pltpu.make_async_copy(out_bufs[b], out_hbm.at[0], wsem.at[b]).wait()

    def compute(b):
        in_ref = in_bufs[b]
        out_ref = out_bufs[b]

        @plsc.parallel_loop(0, OUT_VECS, 1, unroll=8)
        def _(i):
            sl = pl.ds(i * 16, 16)
            r = patr_v[sl]
            cs = pcs_v[sl]
            co = pco_v[sl]
            vals = plsc.load_gather(in_ref, [r, cs])
            plsc.store_scatter(out_ref, [r, co], vals)

    rd_start(0, 0)

    def pair(t, carry):
        for b in (0, 1):
            g = 2 * t + b

            @pl.when(g + 1 < BLK_PER_W)
            def _():
                rd_start(g + 1, 1 - b)

            rd_wait(b)

            @pl.when(g >= 2)
            def _():
                wr_wait(b)

            compute(b)
            wr_start(g, b)
        return carry

    lax.fori_loop(0, BLK_PER_W // 2, pair, 0, unroll=False)
    wr_wait(0)
    wr_wait(1)


@jax.jit
def kernel(x, selected_bands):
    x3 = x.reshape(NBLK, BLK_ROWS, NUM_BANDS_K)
    sel = selected_bands.astype(jnp.int32)
    q = np.arange(OUT_WORDS)
    patr = jnp.asarray(q // TOP_K_K, jnp.int32)
    pco = jnp.asarray(q % TOP_K_K, jnp.int32)
    pcs = sel[q % TOP_K_K]

    mesh = plsc.VectorSubcoreMesh(
        core_axis_name="c", subcore_axis_name="s", num_cores=NC,
        num_subcores=NS)
    fn = pl.kernel(
        _sc_body,
        out_type=jax.ShapeDtypeStruct((NBLK, BLK_ROWS, TOP_K_K), jnp.float32),
        mesh=mesh,
        compiler_params=pltpu.CompilerParams(needs_layout_passes=False),
        scratch_types=[
            pltpu.VMEM((OUT_WORDS,), jnp.int32),
            pltpu.VMEM((OUT_WORDS,), jnp.int32),
            pltpu.VMEM((OUT_WORDS,), jnp.int32),
            pltpu.VMEM((BLK_ROWS, NUM_BANDS_K), jnp.float32),
            pltpu.VMEM((BLK_ROWS, NUM_BANDS_K), jnp.float32),
            pltpu.VMEM((BLK_ROWS, TOP_K_K), jnp.float32),
            pltpu.VMEM((BLK_ROWS, TOP_K_K), jnp.float32),
            pltpu.SemaphoreType.DMA((2,)),
            pltpu.SemaphoreType.DMA((2,)),
        ],
    )
    out = fn(x3, patr, pcs, pco)
    return out.reshape(16, 128, 128, TOP_K_K)


# speedup vs baseline: 13.6772x; 5.1156x over previous
import jax
import jax.numpy as jnp
import numpy as np
from jax import lax
from jax.experimental import pallas as pl
from jax.experimental.pallas import tpu as pltpu
from jax.experimental.pallas import tpu_sc as plsc

NUM_BANDS_K = 200
TOP_K_K = 30
D = 128
NROWS_OUT = 16 * TOP_K_K * 128
NROWS_IN = 16 * 128 * NUM_BANDS_K
NC, NS = 2, 16
NW = NC * NS
CHUNK = 128
CH_PER_W = NROWS_OUT // (NW * CHUNK)
NCHUNKS = NW * CH_PER_W


def _sc_body(tab_hbm, idx_hbm, out_hbm, idx_v, in_v0, in_v1, rsem, wsem):
    in_bufs = (in_v0, in_v1)
    c = lax.axis_index("c")
    s = lax.axis_index("s")
    wid = s * NC + c

    pltpu.sync_copy(idx_hbm.at[wid], idx_v)
    ch_base = wid * CH_PER_W

    def gather_start(g, b):
        pltpu.async_copy(tab_hbm.at[idx_v.at[g]], in_bufs[b], rsem.at[b])

    def gather_wait(b):
        pltpu.make_async_copy(tab_hbm.at[idx_v.at[0]], in_bufs[b],
                              rsem.at[b]).wait()

    def wr_start(g, b):
        pltpu.async_copy(in_bufs[b], out_hbm.at[ch_base + g], wsem.at[b])

    def wr_wait(b):
        pltpu.make_async_copy(in_bufs[b], out_hbm.at[0], wsem.at[b]).wait()

    gather_start(0, 0)
    for g in range(CH_PER_W):
        b = g % 2
        gather_wait(b)
        wr_start(g, b)
        if g + 1 < CH_PER_W:
            nb = (g + 1) % 2
            if g >= 1:
                wr_wait(nb)
            gather_start(g + 1, nb)
    wr_wait((CH_PER_W - 2) % 2)
    wr_wait((CH_PER_W - 1) % 2)


@jax.jit
def kernel(x, selected_bands):
    table = jnp.transpose(x, (0, 1, 3, 2)).reshape(NROWS_IN, D)
    sel = selected_bands.astype(jnp.int32)

    m = np.arange(NROWS_OUT)
    base = ((m // 3840) * 128 + (m % 128)) * NUM_BANDS_K
    idx = (jnp.asarray(base, jnp.int32) + sel[(m // 128) % TOP_K_K]
           ).reshape(NW, CH_PER_W, CHUNK)

    mesh = plsc.VectorSubcoreMesh(
        core_axis_name="c", subcore_axis_name="s", num_cores=NC,
        num_subcores=NS)
    fn = pl.kernel(
        _sc_body,
        out_type=jax.ShapeDtypeStruct((NCHUNKS, CHUNK, D), jnp.float32),
        mesh=mesh,
        compiler_params=pltpu.CompilerParams(needs_layout_passes=False),
        scratch_types=[
            pltpu.VMEM((CH_PER_W, CHUNK), jnp.int32),
            pltpu.VMEM((CHUNK, D), jnp.float32),
            pltpu.VMEM((CHUNK, D), jnp.float32),
            pltpu.SemaphoreType.DMA((2,)),
            pltpu.SemaphoreType.DMA((2,)),
        ],
    )
    out = fn(table, idx)
    out4 = out.reshape(16, TOP_K_K, 128, 128)
    return jnp.transpose(out4, (0, 2, 3, 1))


# speedup vs baseline: 22.8145x; 1.6681x over previous
import jax
import jax.numpy as jnp
from jax import lax
from jax.experimental import pallas as pl
from jax.experimental.pallas import tpu as pltpu
from jax.experimental.pallas import tpu_sc as plsc

NUM_BANDS_K = 200
TOP_K_K = 30
D = 128
NROWS_IN = 16 * 128 * NUM_BANDS_K
NC, NS = 2, 16
NW = NC * NS
CHUNK = 128
CH_PER_W = 15
NCHUNKS = NW * CH_PER_W
NBUF = 3


def _sc_body(tab_hbm, sel_hbm, out_hbm, sel_v, idx_v, b0, b1, b2, rsem, wsem):
    bufs = (b0, b1, b2)
    c = lax.axis_index("c")
    s = lax.axis_index("s")
    wid = s * NC + c

    pltpu.sync_copy(sel_hbm, sel_v)
    ch_base = wid * CH_PER_W

    iotas = [lax.iota(jnp.int32, 16) + (16 * k) for k in range(8)]

    def build_idx(g):
        u = wid * CH_PER_W + g
        bimg = lax.div(u, TOP_K_K)
        sband = lax.rem(u, TOP_K_K)
        selv = plsc.load_gather(
            sel_v, [lax.broadcast_in_dim(sband, (16,), ())])
        base = bimg * (128 * NUM_BANDS_K)
        for k in range(8):
            idx_v[pl.ds(g * CHUNK + 16 * k, 16)] = (
                base + iotas[k] * NUM_BANDS_K + selv)

    def gather_start(g, b):
        pltpu.async_copy(tab_hbm.at[idx_v.at[pl.ds(g * CHUNK, CHUNK)]],
                         bufs[b], rsem.at[b])

    def gather_wait(b):
        pltpu.make_async_copy(tab_hbm.at[idx_v.at[pl.ds(0, CHUNK)]],
                              bufs[b], rsem.at[b]).wait()

    def wr_start(g, b):
        pltpu.async_copy(bufs[b], out_hbm.at[ch_base + g], wsem.at[b])

    def wr_wait(b):
        pltpu.make_async_copy(bufs[b], out_hbm.at[0], wsem.at[b]).wait()

    for g in range(CH_PER_W):
        build_idx(g)
        if g < 2:
            gather_start(g, g % NBUF)
    for g in range(CH_PER_W):
        b = g % NBUF
        gather_wait(b)
        wr_start(g, b)
        nxt = g + 2
        if nxt < CH_PER_W:
            nb = nxt % NBUF
            if nxt >= NBUF:
                wr_wait(nb)
            gather_start(nxt, nb)
    wr_wait((CH_PER_W - 3) % NBUF)
    wr_wait((CH_PER_W - 2) % NBUF)
    wr_wait((CH_PER_W - 1) % NBUF)


@jax.jit
def kernel(x, selected_bands):
    table = jnp.transpose(x, (0, 1, 3, 2)).reshape(NROWS_IN, D)
    sel = jnp.pad(selected_bands.astype(jnp.int32), (0, 2))

    mesh = plsc.VectorSubcoreMesh(
        core_axis_name="c", subcore_axis_name="s", num_cores=NC,
        num_subcores=NS)
    fn = pl.kernel(
        _sc_body,
        out_type=jax.ShapeDtypeStruct((NCHUNKS, CHUNK, D), jnp.float32),
        mesh=mesh,
        compiler_params=pltpu.CompilerParams(needs_layout_passes=False),
        scratch_types=[
            pltpu.VMEM((32,), jnp.int32),
            pltpu.VMEM((CH_PER_W * CHUNK,), jnp.int32),
            pltpu.VMEM((CHUNK, D), jnp.float32),
            pltpu.VMEM((CHUNK, D), jnp.float32),
            pltpu.VMEM((CHUNK, D), jnp.float32),
            pltpu.SemaphoreType.DMA((NBUF,)),
            pltpu.SemaphoreType.DMA((NBUF,)),
        ],
    )
    out = fn(table, sel)
    out4 = out.reshape(16, TOP_K_K, 128, 128)
    return jnp.transpose(out4, (0, 2, 3, 1))


# speedup vs baseline: 23.0902x; 1.0121x over previous
import jax
import jax.numpy as jnp
from jax import lax
from jax.experimental import pallas as pl
from jax.experimental.pallas import tpu as pltpu
from jax.experimental.pallas import tpu_sc as plsc

NUM_BANDS_K = 200
TOP_K_K = 30
D = 128
NROWS_IN = 16 * 128 * NUM_BANDS_K
NC, NS = 2, 16
NW = NC * NS
CHUNK = 128
CH_PER_W = 15
NCHUNKS = NW * CH_PER_W
NBUF = 4


def _sc_body(tab_hbm, sel_hbm, out_hbm, sel_v, idx_v, b0, b1, b2, b3,
             rsem, wsem):
    bufs = (b0, b1, b2, b3)
    c = lax.axis_index("c")
    s = lax.axis_index("s")
    wid = s * NC + c

    pltpu.sync_copy(sel_hbm, sel_v)
    ch_base = wid * CH_PER_W

    iotas = [lax.iota(jnp.int32, 16) + (16 * k) for k in range(8)]

    def build_idx(g):
        u = wid * CH_PER_W + g
        bimg = lax.div(u, TOP_K_K)
        sband = lax.rem(u, TOP_K_K)
        selv = plsc.load_gather(
            sel_v, [lax.broadcast_in_dim(sband, (16,), ())])
        base = bimg * (128 * NUM_BANDS_K)
        for k in range(8):
            idx_v[pl.ds(g * CHUNK + 16 * k, 16)] = (
                base + iotas[k] * NUM_BANDS_K + selv)

    def gather_start(g, b):
        pltpu.async_copy(tab_hbm.at[idx_v.at[pl.ds(g * CHUNK, CHUNK)]],
                         bufs[b], rsem.at[b])

    def gather_wait(b):
        pltpu.make_async_copy(tab_hbm.at[idx_v.at[pl.ds(0, CHUNK)]],
                              bufs[b], rsem.at[b]).wait()

    def wr_start(g, b):
        pltpu.async_copy(bufs[b], out_hbm.at[ch_base + g], wsem.at[b])

    def wr_wait(b):
        pltpu.make_async_copy(bufs[b], out_hbm.at[0], wsem.at[b]).wait()

    LOOKAHEAD = 3
    for g in range(CH_PER_W):
        build_idx(g)
        if g < LOOKAHEAD:
            gather_start(g, g % NBUF)
    for g in range(CH_PER_W):
        b = g % NBUF
        gather_wait(b)
        wr_start(g, b)
        nxt = g + LOOKAHEAD
        if nxt < CH_PER_W:
            nb = nxt % NBUF
            if nxt >= NBUF:
                wr_wait(nb)
            gather_start(nxt, nb)
    for g in range(CH_PER_W - NBUF, CH_PER_W):
        wr_wait(g % NBUF)


@jax.jit
def kernel(x, selected_bands):
    table = jnp.transpose(x, (0, 1, 3, 2)).reshape(NROWS_IN, D)
    sel = selected_bands.astype(jnp.int32)

    mesh = plsc.VectorSubcoreMesh(
        core_axis_name="c", subcore_axis_name="s", num_cores=NC,
        num_subcores=NS)
    fn = pl.kernel(
        _sc_body,
        out_type=jax.ShapeDtypeStruct((NCHUNKS, CHUNK, D), jnp.float32),
        mesh=mesh,
        compiler_params=pltpu.CompilerParams(needs_layout_passes=False),
        scratch_types=[
            pltpu.VMEM((TOP_K_K,), jnp.int32),
            pltpu.VMEM((CH_PER_W * CHUNK,), jnp.int32),
            pltpu.VMEM((CHUNK, D), jnp.float32),
            pltpu.VMEM((CHUNK, D), jnp.float32),
            pltpu.VMEM((CHUNK, D), jnp.float32),
            pltpu.VMEM((CHUNK, D), jnp.float32),
            pltpu.SemaphoreType.DMA((NBUF,)),
            pltpu.SemaphoreType.DMA((NBUF,)),
        ],
    )
    out = fn(table, sel)
    out4 = out.reshape(16, TOP_K_K, 128, 128)
    return jnp.transpose(out4, (0, 2, 3, 1))
